# initial kernel scaffold (unmeasured)
import jax
import jax.numpy as jnp
from jax import lax
from jax.experimental import pallas as pl
from jax.experimental.pallas import tpu as pltpu


def kernel(x, assign, W1, W2):
    T, D = x.shape
    E, _, F = W1.shape

    def body(x_ref, a_ref, w1_ref, w2_ref, out_ref,
             xpeer_ref, apeer_ref, ysend_ref, yret_ref,
             send_sems, recv_sems):
        my_x = lax.axis_index("x")
        my_y = lax.axis_index("y")
        peer = (1 - my_x, my_y)

        barrier = pltpu.get_barrier_semaphore()
        pl.semaphore_signal(barrier, inc=1, device_id=peer,
                            device_id_type=pl.DeviceIdType.MESH)
        pl.semaphore_wait(barrier, 1)

        rdma_x = pltpu.make_async_remote_copy(
            src_ref=x_ref, dst_ref=xpeer_ref,
            send_sem=send_sems.at[0], recv_sem=recv_sems.at[0],
            device_id=peer, device_id_type=pl.DeviceIdType.MESH)
        rdma_x.start()
        rdma_a = pltpu.make_async_remote_copy(
            src_ref=a_ref, dst_ref=apeer_ref,
            send_sem=send_sems.at[1], recv_sem=recv_sems.at[1],
            device_id=peer, device_id_type=pl.DeviceIdType.MESH)
        rdma_a.start()

        def moe(tokens_bf16, assign_vec):
            acc = jnp.zeros((T, D), jnp.float32)
            for e in range(E):
                gid = my_x * E + e
                w1b = w1_ref[e].astype(jnp.bfloat16)
                w2b = w2_ref[e].astype(jnp.bfloat16)
                h = jnp.maximum(
                    jnp.dot(tokens_bf16, w1b,
                            preferred_element_type=jnp.float32), 0.0
                ).astype(jnp.bfloat16)
                y = jnp.dot(h, w2b, preferred_element_type=jnp.float32)
                acc = acc + jnp.where((assign_vec == gid)[:, None], y, 0.0)
            return acc

        acc = moe(x_ref[...].astype(jnp.bfloat16), a_ref[...])

        rdma_x.wait()
        rdma_a.wait()

        ysend_ref[...] = moe(xpeer_ref[...].astype(jnp.bfloat16),
                             apeer_ref[...])

        rdma_y = pltpu.make_async_remote_copy(
            src_ref=ysend_ref, dst_ref=yret_ref,
            send_sem=send_sems.at[2], recv_sem=recv_sems.at[2],
            device_id=peer, device_id_type=pl.DeviceIdType.MESH)
        rdma_y.start()
        rdma_y.wait()

        out_ref[...] = acc + yret_ref[...]

    return pl.pallas_call(
        body,
        out_shape=jax.ShapeDtypeStruct((T, D), jnp.float32),
        in_specs=[pl.BlockSpec(memory_space=pltpu.VMEM)] * 4,
        out_specs=pl.BlockSpec(memory_space=pltpu.VMEM),
        scratch_shapes=[
            pltpu.VMEM((T, D), jnp.float32),
            pltpu.VMEM((T,), jnp.int32),
            pltpu.VMEM((T, D), jnp.float32),
            pltpu.VMEM((T, D), jnp.float32),
            pltpu.SemaphoreType.DMA((3,)),
            pltpu.SemaphoreType.DMA((3,)),
        ],
        compiler_params=pltpu.CompilerParams(collective_id=0),
    )(x, assign, W1, W2)


# baseline (device time: 131069 ns/iter reference)
import jax
import jax.numpy as jnp
from jax import lax
from jax.experimental import pallas as pl
from jax.experimental.pallas import tpu as pltpu


def kernel(x, assign, W1, W2):
    T, D = x.shape
    E, _, F = W1.shape

    def body(x_ref, a_ref, w1_hbm, w2_hbm, out_ref,
             xpeer_ref, apeer_ref, ysend_ref, yret_ref,
             w1b_ref, w2b_ref, stage_ref, stage2_ref,
             send_sems, recv_sems, copy_sems):
        my_x = lax.axis_index("x")
        my_y = lax.axis_index("y")
        peer = (1 - my_x, my_y)

        barrier = pltpu.get_barrier_semaphore()
        pl.semaphore_signal(barrier, inc=1, device_id=peer,
                            device_id_type=pl.DeviceIdType.MESH)
        pl.semaphore_wait(barrier, 1)

        rdma_x = pltpu.make_async_remote_copy(
            src_ref=x_ref, dst_ref=xpeer_ref,
            send_sem=send_sems.at[0], recv_sem=recv_sems.at[0],
            device_id=peer, device_id_type=pl.DeviceIdType.MESH)
        rdma_x.start()
        rdma_a = pltpu.make_async_remote_copy(
            src_ref=a_ref, dst_ref=apeer_ref,
            send_sem=send_sems.at[1], recv_sem=recv_sems.at[1],
            device_id=peer, device_id_type=pl.DeviceIdType.MESH)
        rdma_a.start()

        for e in range(E):
            c1 = pltpu.make_async_copy(w1_hbm.at[e], stage_ref,
                                       copy_sems.at[0])
            c2 = pltpu.make_async_copy(w2_hbm.at[e], stage2_ref,
                                       copy_sems.at[1])
            c1.start()
            c2.start()
            c1.wait()
            w1b_ref[e] = stage_ref[...].astype(jnp.bfloat16)
            c2.wait()
            w2b_ref[e] = stage2_ref[...].astype(jnp.bfloat16)

        def moe(tokens_bf16, assign_vec):
            acc = jnp.zeros((T, D), jnp.float32)
            for e in range(E):
                gid = my_x * E + e
                h = jnp.maximum(
                    jnp.dot(tokens_bf16, w1b_ref[e],
                            preferred_element_type=jnp.float32), 0.0
                ).astype(jnp.bfloat16)
                y = jnp.dot(h, w2b_ref[e],
                            preferred_element_type=jnp.float32)
                acc = acc + jnp.where(assign_vec == gid, y, 0.0)
            return acc

        acc = moe(x_ref[...].astype(jnp.bfloat16), a_ref[...])

        rdma_x.wait()
        rdma_a.wait()

        ysend_ref[...] = moe(xpeer_ref[...].astype(jnp.bfloat16),
                             apeer_ref[...])

        rdma_y = pltpu.make_async_remote_copy(
            src_ref=ysend_ref, dst_ref=yret_ref,
            send_sem=send_sems.at[2], recv_sem=recv_sems.at[2],
            device_id=peer, device_id_type=pl.DeviceIdType.MESH)
        rdma_y.start()
        rdma_y.wait()

        out_ref[...] = acc + yret_ref[...]

    return pl.pallas_call(
        body,
        out_shape=jax.ShapeDtypeStruct((T, D), jnp.float32),
        in_specs=[
            pl.BlockSpec(memory_space=pltpu.VMEM),
            pl.BlockSpec(memory_space=pltpu.VMEM),
            pl.BlockSpec(memory_space=pltpu.MemorySpace.HBM),
            pl.BlockSpec(memory_space=pltpu.MemorySpace.HBM),
        ],
        out_specs=pl.BlockSpec(memory_space=pltpu.VMEM),
        scratch_shapes=[
            pltpu.VMEM((T, D), jnp.float32),
            pltpu.VMEM((T, 1), jnp.int32),
            pltpu.VMEM((T, D), jnp.float32),
            pltpu.VMEM((T, D), jnp.float32),
            pltpu.VMEM((E, D, F), jnp.bfloat16),
            pltpu.VMEM((E, F, D), jnp.bfloat16),
            pltpu.VMEM((D, F), jnp.float32),
            pltpu.VMEM((F, D), jnp.float32),
            pltpu.SemaphoreType.DMA((3,)),
            pltpu.SemaphoreType.DMA((3,)),
            pltpu.SemaphoreType.DMA((2,)),
        ],
        compiler_params=pltpu.CompilerParams(
            collective_id=0,
            vmem_limit_bytes=100 * 1024 * 1024,
        ),
    )(x, assign.reshape(T, 1), W1, W2)


# device time: 77013 ns/iter; 1.7019x vs baseline; 1.7019x over previous
import jax
import jax.numpy as jnp
from jax import lax
from jax.experimental import pallas as pl
from jax.experimental.pallas import tpu as pltpu

N_CHUNK = 2


def kernel(x, assign, W1, W2):
    T, D = x.shape
    E, _, F = W1.shape
    C = T // N_CHUNK

    def body(x_ref, a_ref, w1_hbm, w2_hbm, out_ref,
             xsend_ref, xpeer_ref, apeer_ref, ysend_ref, yret_ref,
             w1b_ref, w2b_ref, stage_ref, stage2_ref,
             send_sems, recv_sems, copy_sems):
        my_x = lax.axis_index("x")
        my_y = lax.axis_index("y")
        peer = (1 - my_x, my_y)

        barrier = pltpu.get_barrier_semaphore()
        pl.semaphore_signal(barrier, inc=1, device_id=peer,
                            device_id_type=pl.DeviceIdType.MESH)
        pl.semaphore_wait(barrier, 1)

        xsend_ref[...] = x_ref[...].astype(jnp.bfloat16)
        rdma_x = pltpu.make_async_remote_copy(
            src_ref=xsend_ref, dst_ref=xpeer_ref,
            send_sem=send_sems.at[0], recv_sem=recv_sems.at[0],
            device_id=peer, device_id_type=pl.DeviceIdType.MESH)
        rdma_x.start()
        rdma_a = pltpu.make_async_remote_copy(
            src_ref=a_ref, dst_ref=apeer_ref,
            send_sem=send_sems.at[1], recv_sem=recv_sems.at[1],
            device_id=peer, device_id_type=pl.DeviceIdType.MESH)
        rdma_a.start()

        for e in range(E):
            c1 = pltpu.make_async_copy(w1_hbm.at[e], stage_ref,
                                       copy_sems.at[0])
            c2 = pltpu.make_async_copy(w2_hbm.at[e], stage2_ref,
                                       copy_sems.at[1])
            c1.start()
            c2.start()
            c1.wait()
            w1b_ref[e] = stage_ref[...].astype(jnp.bfloat16)
            c2.wait()
            w2b_ref[e] = stage2_ref[...].astype(jnp.bfloat16)

        def moe(tokens_bf16, assign_vec):
            rows = tokens_bf16.shape[0]
            acc = jnp.zeros((rows, D), jnp.float32)
            for e in range(E):
                gid = my_x * E + e
                h = jnp.maximum(
                    jnp.dot(tokens_bf16, w1b_ref[e],
                            preferred_element_type=jnp.float32), 0.0
                ).astype(jnp.bfloat16)
                y = jnp.dot(h, w2b_ref[e],
                            preferred_element_type=jnp.float32)
                acc = acc + jnp.where(assign_vec == gid, y, 0.0)
            return acc

        out_ref[...] = moe(x_ref[...].astype(jnp.bfloat16), a_ref[...])

        rdma_x.wait()
        rdma_a.wait()

        rets = []
        for c in range(N_CHUNK):
            sl = pl.ds(c * C, C)
            ysend_ref[sl, :] = moe(xpeer_ref[sl, :],
                                   apeer_ref[sl, :]).astype(jnp.bfloat16)
            rdma_y = pltpu.make_async_remote_copy(
                src_ref=ysend_ref.at[sl, :], dst_ref=yret_ref.at[sl, :],
                send_sem=send_sems.at[2 + c], recv_sem=recv_sems.at[2 + c],
                device_id=peer, device_id_type=pl.DeviceIdType.MESH)
            rdma_y.start()
            rets.append(rdma_y)
        for r in rets:
            r.wait()

        out_ref[...] += yret_ref[...].astype(jnp.float32)

    return pl.pallas_call(
        body,
        out_shape=jax.ShapeDtypeStruct((T, D), jnp.float32),
        in_specs=[
            pl.BlockSpec(memory_space=pltpu.VMEM),
            pl.BlockSpec(memory_space=pltpu.VMEM),
            pl.BlockSpec(memory_space=pltpu.MemorySpace.HBM),
            pl.BlockSpec(memory_space=pltpu.MemorySpace.HBM),
        ],
        out_specs=pl.BlockSpec(memory_space=pltpu.VMEM),
        scratch_shapes=[
            pltpu.VMEM((T, D), jnp.bfloat16),
            pltpu.VMEM((T, D), jnp.bfloat16),
            pltpu.VMEM((T, 1), jnp.int32),
            pltpu.VMEM((T, D), jnp.bfloat16),
            pltpu.VMEM((T, D), jnp.bfloat16),
            pltpu.VMEM((E, D, F), jnp.bfloat16),
            pltpu.VMEM((E, F, D), jnp.bfloat16),
            pltpu.VMEM((D, F), jnp.float32),
            pltpu.VMEM((F, D), jnp.float32),
            pltpu.SemaphoreType.DMA((2 + N_CHUNK,)),
            pltpu.SemaphoreType.DMA((2 + N_CHUNK,)),
            pltpu.SemaphoreType.DMA((2,)),
        ],
        compiler_params=pltpu.CompilerParams(
            collective_id=0,
            vmem_limit_bytes=100 * 1024 * 1024,
        ),
    )(x, assign.reshape(T, 1), W1, W2)


# device time: 69965 ns/iter; 1.8734x vs baseline; 1.1007x over previous
import jax
import jax.numpy as jnp
from jax import lax
from jax.experimental import pallas as pl
from jax.experimental.pallas import tpu as pltpu


def kernel(x, assign, W1, W2):
    T, D = x.shape
    E, _, F = W1.shape
    H = T // 2

    def body(x_ref, a_ref, w1_hbm, w2_hbm, out_ref,
             xsend_ref, xpeer_ref, apeer_ref, ysend_ref, yret_ref,
             hsend_ref, hfar_ref,
             w1b_ref, w2b_ref, stage_ref, stage2_ref,
             send_sems, recv_sems, copy_sems):
        my_x = lax.axis_index("x")
        my_y = lax.axis_index("y")
        xpeer = (1 - my_x, my_y)
        ypeer = (my_x, 1 - my_y)

        barrier = pltpu.get_barrier_semaphore()
        for nbr in (xpeer, ypeer):
            pl.semaphore_signal(barrier, inc=1, device_id=nbr,
                                device_id_type=pl.DeviceIdType.MESH)
        pl.semaphore_wait(barrier, 2)

        mine = pl.ds(my_y * H, H)

        xsend_ref[...] = x_ref[mine, :].astype(jnp.bfloat16)
        rdma_x = pltpu.make_async_remote_copy(
            src_ref=xsend_ref, dst_ref=xpeer_ref,
            send_sem=send_sems.at[0], recv_sem=recv_sems.at[0],
            device_id=xpeer, device_id_type=pl.DeviceIdType.MESH)
        rdma_x.start()
        rdma_a = pltpu.make_async_remote_copy(
            src_ref=a_ref.at[mine, :], dst_ref=apeer_ref,
            send_sem=send_sems.at[1], recv_sem=recv_sems.at[1],
            device_id=xpeer, device_id_type=pl.DeviceIdType.MESH)
        rdma_a.start()

        for e in range(E):
            c1 = pltpu.make_async_copy(w1_hbm.at[e], stage_ref,
                                       copy_sems.at[0])
            c2 = pltpu.make_async_copy(w2_hbm.at[e], stage2_ref,
                                       copy_sems.at[1])
            c1.start()
            c2.start()
            c1.wait()
            w1b_ref[e] = stage_ref[...].astype(jnp.bfloat16)
            c2.wait()
            w2b_ref[e] = stage2_ref[...].astype(jnp.bfloat16)

        def moe(tokens_bf16, assign_vec):
            rows = tokens_bf16.shape[0]
            acc = jnp.zeros((rows, D), jnp.float32)
            for e in range(E):
                gid = my_x * E + e
                h = jnp.maximum(
                    jnp.dot(tokens_bf16, w1b_ref[e],
                            preferred_element_type=jnp.float32), 0.0
                ).astype(jnp.bfloat16)
                y = jnp.dot(h, w2b_ref[e],
                            preferred_element_type=jnp.float32)
                acc = acc + jnp.where(assign_vec == gid, y, 0.0)
            return acc

        own_part = moe(x_ref[mine, :].astype(jnp.bfloat16), a_ref[mine, :])

        rdma_x.wait()
        rdma_a.wait()

        ysend_ref[...] = moe(xpeer_ref[...], apeer_ref[...]).astype(
            jnp.bfloat16)
        rdma_y = pltpu.make_async_remote_copy(
            src_ref=ysend_ref, dst_ref=yret_ref,
            send_sem=send_sems.at[2], recv_sem=recv_sems.at[2],
            device_id=xpeer, device_id_type=pl.DeviceIdType.MESH)
        rdma_y.start()
        rdma_y.wait()

        full_half = own_part + yret_ref[...].astype(jnp.float32)
        out_ref[mine, :] = full_half
        hsend_ref[...] = full_half.astype(jnp.bfloat16)
        rdma_h = pltpu.make_async_remote_copy(
            src_ref=hsend_ref, dst_ref=hfar_ref,
            send_sem=send_sems.at[3], recv_sem=recv_sems.at[3],
            device_id=ypeer, device_id_type=pl.DeviceIdType.MESH)
        rdma_h.start()
        rdma_h.wait()

        out_ref[pl.ds((1 - my_y) * H, H), :] = hfar_ref[...].astype(
            jnp.float32)

    return pl.pallas_call(
        body,
        out_shape=jax.ShapeDtypeStruct((T, D), jnp.float32),
        in_specs=[
            pl.BlockSpec(memory_space=pltpu.VMEM),
            pl.BlockSpec(memory_space=pltpu.VMEM),
            pl.BlockSpec(memory_space=pltpu.MemorySpace.HBM),
            pl.BlockSpec(memory_space=pltpu.MemorySpace.HBM),
        ],
        out_specs=pl.BlockSpec(memory_space=pltpu.VMEM),
        scratch_shapes=[
            pltpu.VMEM((H, D), jnp.bfloat16),
            pltpu.VMEM((H, D), jnp.bfloat16),
            pltpu.VMEM((H, 1), jnp.int32),
            pltpu.VMEM((H, D), jnp.bfloat16),
            pltpu.VMEM((H, D), jnp.bfloat16),
            pltpu.VMEM((H, D), jnp.bfloat16),
            pltpu.VMEM((H, D), jnp.bfloat16),
            pltpu.VMEM((E, D, F), jnp.bfloat16),
            pltpu.VMEM((E, F, D), jnp.bfloat16),
            pltpu.VMEM((D, F), jnp.float32),
            pltpu.VMEM((F, D), jnp.float32),
            pltpu.SemaphoreType.DMA((4,)),
            pltpu.SemaphoreType.DMA((4,)),
            pltpu.SemaphoreType.DMA((2,)),
        ],
        compiler_params=pltpu.CompilerParams(
            collective_id=0,
            vmem_limit_bytes=100 * 1024 * 1024,
        ),
    )(x, assign.reshape(T, 1), W1, W2)


# device time: 53912 ns/iter; 2.4312x vs baseline; 1.2978x over previous
import jax
import jax.numpy as jnp
from jax import lax
from jax.experimental import pallas as pl
from jax.experimental.pallas import tpu as pltpu

N_CHUNK = 2


def kernel(x, assign, W1, W2):
    T, D = x.shape
    E, _, F = W1.shape
    H = T // 2
    C = H // N_CHUNK

    def body(x_ref, a_ref, w1_hbm, w2_hbm, out_ref,
             xsend_ref, xpeer_ref, apeer_ref, ysend_ref, yret_ref,
             hsend_ref, hfar_ref,
             w1b_ref, w2b_ref, stage_ref, stage2_ref,
             send_sems, recv_sems, copy_sems):
        my_x = lax.axis_index("x")
        my_y = lax.axis_index("y")
        xpeer = (1 - my_x, my_y)
        ypeer = (my_x, 1 - my_y)

        barrier = pltpu.get_barrier_semaphore()
        for nbr in (xpeer, ypeer):
            pl.semaphore_signal(barrier, inc=1, device_id=nbr,
                                device_id_type=pl.DeviceIdType.MESH)
        pl.semaphore_wait(barrier, 2)

        SA = N_CHUNK
        SR = N_CHUNK + 1
        SH = 2 * N_CHUNK + 1

        rdma_a = pltpu.make_async_remote_copy(
            src_ref=a_ref.at[pl.ds(my_y * H, H), :], dst_ref=apeer_ref,
            send_sem=send_sems.at[SA], recv_sem=recv_sems.at[SA],
            device_id=xpeer, device_id_type=pl.DeviceIdType.MESH)
        rdma_a.start()
        rdma_x = []
        for c in range(N_CHUNK):
            sl = pl.ds(c * C, C)
            xsend_ref[sl, :] = x_ref[pl.ds(my_y * H + c * C, C), :].astype(
                jnp.bfloat16)
            r = pltpu.make_async_remote_copy(
                src_ref=xsend_ref.at[sl, :], dst_ref=xpeer_ref.at[sl, :],
                send_sem=send_sems.at[c], recv_sem=recv_sems.at[c],
                device_id=xpeer, device_id_type=pl.DeviceIdType.MESH)
            r.start()
            rdma_x.append(r)

        for e in range(E):
            c1 = pltpu.make_async_copy(w1_hbm.at[e], stage_ref,
                                       copy_sems.at[0])
            c2 = pltpu.make_async_copy(w2_hbm.at[e], stage2_ref,
                                       copy_sems.at[1])
            c1.start()
            c2.start()
            c1.wait()
            w1b_ref[e] = stage_ref[...].astype(jnp.bfloat16)
            c2.wait()
            w2b_ref[e] = stage2_ref[...].astype(jnp.bfloat16)

        def moe(tokens_bf16, assign_vec):
            rows = tokens_bf16.shape[0]
            acc = jnp.zeros((rows, D), jnp.float32)
            for e in range(E):
                gid = my_x * E + e
                h = jnp.maximum(
                    jnp.dot(tokens_bf16, w1b_ref[e],
                            preferred_element_type=jnp.float32), 0.0
                ).astype(jnp.bfloat16)
                y = jnp.dot(h, w2b_ref[e],
                            preferred_element_type=jnp.float32)
                acc = acc + jnp.where(assign_vec == gid, y, 0.0)
            return acc

        rdma_a.wait()
        rdma_ret = []
        for c in range(N_CHUNK):
            sl = pl.ds(c * C, C)
            rdma_x[c].wait()
            ysend_ref[sl, :] = moe(xpeer_ref[sl, :],
                                   apeer_ref[sl, :]).astype(jnp.bfloat16)
            r = pltpu.make_async_remote_copy(
                src_ref=ysend_ref.at[sl, :], dst_ref=yret_ref.at[sl, :],
                send_sem=send_sems.at[SR + c], recv_sem=recv_sems.at[SR + c],
                device_id=xpeer, device_id_type=pl.DeviceIdType.MESH)
            r.start()
            rdma_ret.append(r)

        rdma_h = []
        for c in range(N_CHUNK):
            sl = pl.ds(c * C, C)
            own_c = moe(x_ref[pl.ds(my_y * H + c * C, C), :].astype(
                jnp.bfloat16), a_ref[pl.ds(my_y * H + c * C, C), :])
            rdma_ret[c].wait()
            full_c = own_c + yret_ref[sl, :].astype(jnp.float32)
            out_ref[pl.ds(my_y * H + c * C, C), :] = full_c
            hsend_ref[sl, :] = full_c.astype(jnp.bfloat16)
            r = pltpu.make_async_remote_copy(
                src_ref=hsend_ref.at[sl, :], dst_ref=hfar_ref.at[sl, :],
                send_sem=send_sems.at[SH + c], recv_sem=recv_sems.at[SH + c],
                device_id=ypeer, device_id_type=pl.DeviceIdType.MESH)
            r.start()
            rdma_h.append(r)

        for c in range(N_CHUNK):
            rdma_h[c].wait()
            out_ref[pl.ds((1 - my_y) * H + c * C, C), :] = (
                hfar_ref[pl.ds(c * C, C), :].astype(jnp.float32))

    n_sems = 3 * N_CHUNK + 1
    return pl.pallas_call(
        body,
        out_shape=jax.ShapeDtypeStruct((T, D), jnp.float32),
        in_specs=[
            pl.BlockSpec(memory_space=pltpu.VMEM),
            pl.BlockSpec(memory_space=pltpu.VMEM),
            pl.BlockSpec(memory_space=pltpu.MemorySpace.HBM),
            pl.BlockSpec(memory_space=pltpu.MemorySpace.HBM),
        ],
        out_specs=pl.BlockSpec(memory_space=pltpu.VMEM),
        scratch_shapes=[
            pltpu.VMEM((H, D), jnp.bfloat16),
            pltpu.VMEM((H, D), jnp.bfloat16),
            pltpu.VMEM((H, 1), jnp.int32),
            pltpu.VMEM((H, D), jnp.bfloat16),
            pltpu.VMEM((H, D), jnp.bfloat16),
            pltpu.VMEM((H, D), jnp.bfloat16),
            pltpu.VMEM((H, D), jnp.bfloat16),
            pltpu.VMEM((E, D, F), jnp.bfloat16),
            pltpu.VMEM((E, F, D), jnp.bfloat16),
            pltpu.VMEM((D, F), jnp.float32),
            pltpu.VMEM((F, D), jnp.float32),
            pltpu.SemaphoreType.DMA((n_sems,)),
            pltpu.SemaphoreType.DMA((n_sems,)),
            pltpu.SemaphoreType.DMA((2,)),
        ],
        compiler_params=pltpu.CompilerParams(
            collective_id=0,
            vmem_limit_bytes=100 * 1024 * 1024,
        ),
    )(x, assign.reshape(T, 1), W1, W2)
